# initial kernel scaffold (unmeasured)
import jax
import jax.numpy as jnp
from jax import lax
from jax.experimental import pallas as pl
from jax.experimental.pallas import tpu as pltpu


def kernel(
    x,
):
    def body(*refs):
        pass

    out_shape = jax.ShapeDtypeStruct(..., jnp.float32)
    return pl.pallas_call(body, out_shape=out_shape)(...)


# baseline (device time: 19870 ns/iter reference)
import jax
import jax.numpy as jnp
from jax import lax
from jax.experimental import pallas as pl
from jax.experimental.pallas import tpu as pltpu

N_DEV = 4


def kernel(x):
    m, n_total = x.shape
    blk = n_total // N_DEV
    out_rows = N_DEV * m

    def body(x_ref, out_ref, send_buf, send_sems, recv_sems):
        me = lax.axis_index("i")

        send_buf[:, :] = x_ref[:, :].astype(jnp.bfloat16)

        barrier_sem = pltpu.get_barrier_semaphore()
        for o in range(1, N_DEV):
            pl.semaphore_signal(
                barrier_sem,
                inc=1,
                device_id=((me + o) % N_DEV,),
                device_id_type=pl.DeviceIdType.MESH,
            )
        pl.semaphore_wait(barrier_sem, N_DEV - 1)

        sends = []
        for o in range(1, N_DEV):
            t = (me + o) % N_DEV
            rdma = pltpu.make_async_remote_copy(
                src_ref=send_buf.at[:, pl.ds(t * blk, blk)],
                dst_ref=out_ref.at[pl.ds(me * m, m), :],
                send_sem=send_sems.at[o],
                recv_sem=recv_sems.at[o],
                device_id=(t,),
                device_id_type=pl.DeviceIdType.MESH,
            )
            rdma.start()
            sends.append(rdma)

        out_ref[pl.ds(me * m, m), :] = send_buf[:, pl.ds(me * blk, blk)]

        for o in range(1, N_DEV):
            s = (me - o) % N_DEV
            recv = pltpu.make_async_remote_copy(
                src_ref=send_buf.at[:, pl.ds(s * blk, blk)],
                dst_ref=out_ref.at[pl.ds(s * m, m), :],
                send_sem=send_sems.at[o],
                recv_sem=recv_sems.at[o],
                device_id=(s,),
                device_id_type=pl.DeviceIdType.MESH,
            )
            recv.wait_recv()

        for rdma in sends:
            rdma.wait_send()

    return pl.pallas_call(
        body,
        out_shape=jax.ShapeDtypeStruct((out_rows, blk), jnp.bfloat16),
        in_specs=[pl.BlockSpec(memory_space=pltpu.VMEM)],
        out_specs=pl.BlockSpec(memory_space=pltpu.VMEM),
        scratch_shapes=[
            pltpu.VMEM((m, n_total), jnp.bfloat16),
            pltpu.SemaphoreType.DMA((N_DEV,)),
            pltpu.SemaphoreType.DMA((N_DEV,)),
        ],
        compiler_params=pltpu.CompilerParams(collective_id=0),
    )(x)
